# baseline (speedup 1.0000x reference)
import jax
import jax.numpy as jnp
from jax.experimental import pallas as pl
from jax.experimental.pallas import tpu as pltpu

_LANE = 128
_BATCH_TILE = 512


def _pad_axis(a, axis, multiple):
    pad = (-a.shape[axis]) % multiple
    if pad == 0:
        return a
    widths = [(0, 0)] * a.ndim
    widths[axis] = (0, pad)
    return jnp.pad(a, widths)


def _mlp_kernel(x_ref, w1_ref, b1_ref, w2_ref, b2_ref, o_ref):
    xb = x_ref[...].astype(jnp.bfloat16)
    h = jnp.dot(xb, w1_ref[...], preferred_element_type=jnp.float32)
    h = jnp.maximum(h + b1_ref[...], 0.0).astype(jnp.bfloat16)
    out = jnp.dot(h, w2_ref[...], preferred_element_type=jnp.float32)
    o_ref[...] = (out + b2_ref[...]).astype(o_ref.dtype)


def kernel(x, w1, b1, w2, b2):
    B, IN = x.shape
    OUT = w2.shape[1]

    x_p = _pad_axis(x, 1, _LANE)
    w1_p = _pad_axis(_pad_axis(w1, 0, _LANE), 1, _LANE).astype(jnp.bfloat16)
    b1_p = _pad_axis(b1, 1, _LANE)
    w2_p = _pad_axis(_pad_axis(w2, 0, _LANE), 1, _LANE).astype(jnp.bfloat16)
    b2_p = _pad_axis(b2, 1, _LANE)
    IN_P, H_P = w1_p.shape
    OUT_P = w2_p.shape[1]

    tb = min(_BATCH_TILE, max(8, B))
    n_tiles = pl.cdiv(B, tb)
    x_p = _pad_axis(x_p, 0, tb)

    out_p = pl.pallas_call(
        _mlp_kernel,
        out_shape=jax.ShapeDtypeStruct((n_tiles * tb, OUT_P), x.dtype),
        grid=(n_tiles,),
        in_specs=[
            pl.BlockSpec((tb, IN_P), lambda i: (i, 0)),
            pl.BlockSpec((IN_P, H_P), lambda i: (0, 0)),
            pl.BlockSpec((1, H_P), lambda i: (0, 0)),
            pl.BlockSpec((H_P, OUT_P), lambda i: (0, 0)),
            pl.BlockSpec((1, OUT_P), lambda i: (0, 0)),
        ],
        out_specs=pl.BlockSpec((tb, OUT_P), lambda i: (i, 0)),
        compiler_params=pltpu.CompilerParams(
            dimension_semantics=("parallel",),
        ),
    )(x_p, w1_p, b1_p, w2_p, b2_p)
    return out_p[:B, :OUT]


# speedup vs baseline: 1.0552x; 1.0552x over previous
import jax
import jax.numpy as jnp
from jax.experimental import pallas as pl
from jax.experimental.pallas import tpu as pltpu

_LANE = 128
_BATCH_TILE = 2048


def _pad_axis(a, axis, multiple):
    pad = (-a.shape[axis]) % multiple
    if pad == 0:
        return a
    widths = [(0, 0)] * a.ndim
    widths[axis] = (0, pad)
    return jnp.pad(a, widths)


def _mlp_kernel(x_ref, w1_ref, b1_ref, w2_ref, b2_ref, o_ref):
    h = jnp.dot(x_ref[...], w1_ref[...], preferred_element_type=jnp.float32)
    h = jnp.maximum(h + b1_ref[...], 0.0).astype(jnp.bfloat16)
    out = jnp.dot(h, w2_ref[...].astype(jnp.bfloat16),
                  preferred_element_type=jnp.float32)
    o_ref[...] = (out + b2_ref[...]).astype(o_ref.dtype)


def kernel(x, w1, b1, w2, b2):
    B, IN = x.shape
    OUT = w2.shape[1]

    x_p = _pad_axis(x, 1, _LANE)
    w1_p = _pad_axis(_pad_axis(w1, 0, _LANE), 1, _LANE)
    b1_p = _pad_axis(b1, 1, _LANE)
    w2_p = _pad_axis(_pad_axis(w2, 0, _LANE), 1, _LANE)
    b2_p = _pad_axis(b2, 1, _LANE)
    IN_P, H_P = w1_p.shape
    OUT_P = w2_p.shape[1]

    tb = min(_BATCH_TILE, max(8, B))
    n_tiles = pl.cdiv(B, tb)
    x_p = _pad_axis(x_p, 0, tb)

    out_p = pl.pallas_call(
        _mlp_kernel,
        out_shape=jax.ShapeDtypeStruct((n_tiles * tb, OUT_P), x.dtype),
        grid=(n_tiles,),
        in_specs=[
            pl.BlockSpec((tb, IN_P), lambda i: (i, 0)),
            pl.BlockSpec((IN_P, H_P), lambda i: (0, 0)),
            pl.BlockSpec((1, H_P), lambda i: (0, 0)),
            pl.BlockSpec((H_P, OUT_P), lambda i: (0, 0)),
            pl.BlockSpec((1, OUT_P), lambda i: (0, 0)),
        ],
        out_specs=pl.BlockSpec((tb, OUT_P), lambda i: (i, 0)),
        compiler_params=pltpu.CompilerParams(
            dimension_semantics=("parallel",),
            vmem_limit_bytes=64 * 1024 * 1024,
        ),
    )(x_p, w1_p, b1_p, w2_p, b2_p)
    return out_p[:B, :OUT]


# speedup vs baseline: 1.0829x; 1.0263x over previous
import functools

import jax
import jax.numpy as jnp
from jax.experimental import pallas as pl
from jax.experimental.pallas import tpu as pltpu

_TB = 1024


def _pad_axis(a, axis, multiple):
    pad = (-a.shape[axis]) % multiple
    if pad == 0:
        return a
    widths = [(0, 0)] * a.ndim
    widths[axis] = (0, pad)
    return jnp.pad(a, widths)


def _mlp_pipeline_kernel(n_steps, x_hbm, w1_hbm, b1_hbm, w2_hbm, b2_hbm,
                         o_hbm, x_buf, o_buf, w1_v, b1_v, w2_v, b2_v,
                         x_sem, o_sem, w_sem):
    tb = x_buf.shape[1]

    def x_in(slot, step):
        return pltpu.make_async_copy(
            x_hbm.at[pl.ds(step * tb, tb)], x_buf.at[slot], x_sem.at[slot])

    def o_out(slot, step):
        return pltpu.make_async_copy(
            o_buf.at[slot], o_hbm.at[pl.ds(step * tb, tb)], o_sem.at[slot])

    cp_w1 = pltpu.make_async_copy(w1_hbm, w1_v, w_sem.at[0])
    cp_b1 = pltpu.make_async_copy(b1_hbm, b1_v, w_sem.at[1])
    cp_w2 = pltpu.make_async_copy(w2_hbm, w2_v, w_sem.at[2])
    cp_b2 = pltpu.make_async_copy(b2_hbm, b2_v, w_sem.at[3])

    cp_w1.start()
    cp_b1.start()
    x_in(0, 0).start()
    cp_w2.start()
    cp_b2.start()

    @pl.when(n_steps > 1)
    def _():
        x_in(1, 1).start()

    def compute(slot, wait_w2):
        h = jnp.dot(x_buf[slot], w1_v[...],
                    preferred_element_type=jnp.float32)
        h = jnp.maximum(h + b1_v[...], 0.0)
        if wait_w2:
            cp_w2.wait()
            cp_b2.wait()
        out = jnp.dot(h, w2_v[...], preferred_element_type=jnp.float32)
        o_buf[slot] = out + b2_v[...]

    cp_w1.wait()
    cp_b1.wait()
    x_in(0, 0).wait()
    compute(0, True)
    o_out(0, 0).start()

    def body(step, _):
        slot = jax.lax.rem(step, 2)

        @pl.when(step + 1 < n_steps)
        def _():
            x_in(slot ^ 1, step + 1).start()

        x_in(slot, step).wait()

        @pl.when(step >= 2)
        def _():
            o_out(slot, step).wait()

        compute(slot, False)
        o_out(slot, step).start()
        return ()

    jax.lax.fori_loop(1, n_steps, body, ())

    @pl.when(n_steps > 1)
    def _():
        o_out(jax.lax.rem(n_steps - 2, 2), 0).wait()
    o_out(jax.lax.rem(n_steps - 1, 2), 0).wait()


def kernel(x, w1, b1, w2, b2):
    B, IN = x.shape
    OUT = w2.shape[1]

    x_p = _pad_axis(x, 1, 128)
    w1_p = _pad_axis(_pad_axis(w1, 0, 128), 1, 128)
    b1_p = _pad_axis(b1, 1, 128)
    w2_p = _pad_axis(_pad_axis(w2, 0, 128), 1, 128)
    b2_p = _pad_axis(b2, 1, 128)
    IN_P, H_P = w1_p.shape
    OUT_P = w2_p.shape[1]

    tb = _TB if B % _TB == 0 else B
    x_p = _pad_axis(x_p, 0, tb)
    n_steps = x_p.shape[0] // tb

    body = functools.partial(_mlp_pipeline_kernel, n_steps)

    out_p = pl.pallas_call(
        body,
        out_shape=jax.ShapeDtypeStruct((n_steps * tb, OUT_P), x.dtype),
        in_specs=[pl.BlockSpec(memory_space=pltpu.MemorySpace.HBM)] * 5,
        out_specs=pl.BlockSpec(memory_space=pltpu.MemorySpace.HBM),
        scratch_shapes=[
            pltpu.VMEM((2, tb, IN_P), jnp.float32),
            pltpu.VMEM((2, tb, OUT_P), jnp.float32),
            pltpu.VMEM((IN_P, H_P), jnp.float32),
            pltpu.VMEM((1, H_P), jnp.float32),
            pltpu.VMEM((H_P, OUT_P), jnp.float32),
            pltpu.VMEM((1, OUT_P), jnp.float32),
            pltpu.SemaphoreType.DMA((2,)),
            pltpu.SemaphoreType.DMA((2,)),
            pltpu.SemaphoreType.DMA((4,)),
        ],
        compiler_params=pltpu.CompilerParams(
            vmem_limit_bytes=64 * 1024 * 1024,
        ),
    )(x_p, w1_p, b1_p, w2_p, b2_p)
    return out_p[:B, :OUT]
